# Initial kernel scaffold; baseline (speedup 1.0000x reference)
import jax
import jax.numpy as jnp
from jax.experimental import pallas as pl


def kernel(feature, percentiles, fc_w, fc_b):
    raise NotImplementedError("write your pallas kernel here")


# speedup vs baseline: 7.2935x; 7.2935x over previous
import functools

import jax
import jax.numpy as jnp
from jax import lax
from jax.experimental import pallas as pl
from jax.experimental.pallas import tpu as pltpu

B = 128
C = 32768
N_PAD = 1024


ROWS_BLK = 16


def _scale_body(x_ref, ks_ref, out_ref):
    x = x_ref[...]
    y = jnp.maximum(x, 0.0)
    ks = ks_ref[...]
    batch_sum = jnp.sum(y, axis=1, keepdims=True)
    yi = pltpu.bitcast(y, jnp.int32)

    def bit_step(i, t):
        cand = t | lax.shift_left(jnp.int32(1), 30 - i)
        cnt = jnp.sum((yi >= cand).astype(jnp.int32), axis=1, keepdims=True)
        return jnp.where(cnt >= ks, cand, t)

    t_bits = lax.fori_loop(0, 31, bit_step, jnp.zeros((ROWS_BLK, 1), jnp.int32))

    gt = yi > t_bits
    n_gt = jnp.sum(gt.astype(jnp.int32), axis=1, keepdims=True)
    sum_gt = jnp.sum(jnp.where(gt, y, 0.0), axis=1, keepdims=True)
    t_val = pltpu.bitcast(t_bits, jnp.float32)
    topk_sum = sum_gt + (ks - n_gt).astype(jnp.float32) * t_val
    out_ref[...] = batch_sum / topk_sum


def _scales(feature, ks):
    return pl.pallas_call(
        _scale_body,
        grid=(B // ROWS_BLK,),
        in_specs=[
            pl.BlockSpec((ROWS_BLK, C), lambda i: (i, 0)),
            pl.BlockSpec((ROWS_BLK, 1), lambda i: (i, 0)),
        ],
        out_specs=pl.BlockSpec((ROWS_BLK, 1), lambda i: (i, 0)),
        out_shape=jax.ShapeDtypeStruct((B, 1), jnp.float32),
    )(feature, ks)


BN = 256
BC = 4096


def _mm_body(x_ref, w_ref, s_ref, b_ref, o_ref):
    j = pl.program_id(1)

    @pl.when(j == 0)
    def _():
        o_ref[...] = jnp.zeros_like(o_ref)

    o_ref[...] += lax.dot_general(
        x_ref[...], w_ref[...],
        dimension_numbers=(((1,), (1,)), ((), ())),
        preferred_element_type=jnp.float32,
    )

    @pl.when(j == pl.num_programs(1) - 1)
    def _():
        o_ref[...] = o_ref[...] * jnp.exp(s_ref[...]) + b_ref[...]


def _logits(feature, fc_w_pad, scale, fc_b_pad):
    return pl.pallas_call(
        _mm_body,
        grid=(N_PAD // BN, C // BC),
        in_specs=[
            pl.BlockSpec((B, BC), lambda i, j: (0, j)),
            pl.BlockSpec((BN, BC), lambda i, j: (i, j)),
            pl.BlockSpec((B, 1), lambda i, j: (0, 0)),
            pl.BlockSpec((1, BN), lambda i, j: (0, i)),
        ],
        out_specs=pl.BlockSpec((B, BN), lambda i, j: (0, i)),
        out_shape=jax.ShapeDtypeStruct((B, N_PAD), jnp.float32),
    )(feature, fc_w_pad, scale, fc_b_pad)


def kernel(feature, percentiles, fc_w, fc_b):
    n_classes = fc_w.shape[0]
    ks = (C - jnp.round(C * percentiles / 100.0).astype(jnp.int32)).reshape(B, 1)
    scale = _scales(feature, ks)
    w_pad = jnp.pad(fc_w, ((0, N_PAD - n_classes), (0, 0)))
    b_pad = jnp.pad(fc_b, (0, N_PAD - n_classes)).reshape(1, N_PAD)
    out = _logits(feature, w_pad, scale, b_pad)
    return out[:, :n_classes]
